# Initial kernel scaffold; baseline (speedup 1.0000x reference)
import functools

import jax
import jax.numpy as jnp
from jax import lax
from jax.experimental import pallas as pl
from jax.experimental.pallas import tpu as pltpu
from jax.experimental.pallas import tpu_sc as plsc

_N = 10000
_E = 320000
_D = 128
_H = 16
_O = 2

_NSUB = 16
_NCORE = 2
_NW = _NSUB * _NCORE
_NP = 10240
_RPT = _NP // _NSUB
_EPW = _E // _NW
_CH = 80
_NCH = _EPW // _CH

_mesh = plsc.VectorSubcoreMesh(core_axis_name="c", subcore_axis_name="s")


@functools.partial(
    pl.kernel,
    out_type=jax.ShapeDtypeStruct((_NW, _NP), jnp.float32),
    mesh=_mesh,
    scratch_types=[
        pltpu.VMEM((_NCH, _CH), jnp.int32),
        pltpu.VMEM((_NP,), jnp.float32),
    ],
)
def _deg_kernel(dstr_hbm, out_hbm, idx_v, acc_v):
    c = lax.axis_index("c")
    s = lax.axis_index("s")
    wid = c * _NSUB + s
    zero16 = jnp.zeros((16,), jnp.float32)

    def zbody(i, carry):
        acc_v[pl.ds(i * 16, 16)] = zero16
        return carry

    lax.fori_loop(0, _NP // 16, zbody, 0)
    pltpu.sync_copy(dstr_hbm.at[wid], idx_v)
    ones16 = jnp.ones((16,), jnp.float32)

    def body(j, carry):
        for k in range(_CH // 16):
            idx = idx_v[j, pl.ds(k * 16, 16)]
            plsc.addupdate_scatter(acc_v, [idx], ones16)
        return carry

    lax.fori_loop(0, _NCH, body, 0)
    pltpu.sync_copy(acc_v, out_hbm.at[wid])


@functools.partial(
    pl.kernel,
    out_type=jax.ShapeDtypeStruct((_NCORE, _NP, _H), jnp.float32),
    mesh=_mesh,
    scratch_types=[
        pltpu.VMEM((_NCH, _CH), jnp.int32),
        pltpu.VMEM((_NCH, _CH), jnp.int32),
        pltpu.VMEM((_CH, _H), jnp.float32),
        pltpu.VMEM((_RPT, _H), jnp.float32),
        pltpu.VMEM_SHARED((_NP, _H), jnp.float32),
        pltpu.SemaphoreType.DMA,
    ],
)
def _agg_kernel(y_hbm, srcr_hbm, dstr_hbm, out_hbm,
                idxs_v, idxd_v, rows_v, zbuf_v, acc_sh, sem):
    c = lax.axis_index("c")
    s = lax.axis_index("s")
    wid = c * _NSUB + s
    zero16 = jnp.zeros((16,), jnp.float32)

    def zbody(i, carry):
        zbuf_v[i, :] = zero16
        return carry

    lax.fori_loop(0, _RPT, zbody, 0)
    pltpu.sync_copy(zbuf_v, acc_sh.at[pl.ds(s * _RPT, _RPT)])
    pltpu.sync_copy(srcr_hbm.at[wid], idxs_v)
    pltpu.sync_copy(dstr_hbm.at[wid], idxd_v)
    plsc.subcore_barrier()

    def body(j, carry):
        pltpu.async_copy(y_hbm.at[idxs_v.at[j]], rows_v, sem).wait()
        pltpu.sync_copy(rows_v, acc_sh.at[idxd_v.at[j]], add=True)
        return carry

    lax.fori_loop(0, _NCH, body, 0)
    plsc.subcore_barrier()
    pltpu.sync_copy(acc_sh.at[pl.ds(s * _RPT, _RPT)],
                    out_hbm.at[c, pl.ds(s * _RPT, _RPT)])


def _prep_body(parts_ref, x_ref, w1_ref, y_ref, inv_ref):
    deg = jnp.sum(parts_ref[...], axis=0) + 1.0
    inv = lax.rsqrt(deg)
    xw = jnp.dot(x_ref[...], w1_ref[...], preferred_element_type=jnp.float32)
    y_ref[...] = xw * inv[:, None]
    inv_ref[...] = jnp.broadcast_to(inv[:, None], (_NP, _H))


_prep = pl.pallas_call(
    _prep_body,
    out_shape=[
        jax.ShapeDtypeStruct((_NP, _H), jnp.float32),
        jax.ShapeDtypeStruct((_NP, _H), jnp.float32),
    ],
)


def _mid_body(p_ref, y_ref, inv_ref, b_ref, w2_ref, y2_ref):
    srec = p_ref[0] + p_ref[1] + y_ref[...]
    h = jnp.maximum(inv_ref[...] * srec + b_ref[...], 0.0)
    y2_ref[...] = jnp.dot(h, w2_ref[...],
                          preferred_element_type=jnp.float32) * inv_ref[...]


_mid = pl.pallas_call(
    _mid_body,
    out_shape=jax.ShapeDtypeStruct((_NP, _H), jnp.float32),
)


def _fin_body(p_ref, y2_ref, inv_ref, b2_ref, wfc_ref, bfc_ref, out_ref):
    srec = p_ref[0] + p_ref[1] + y2_ref[...]
    h = jnp.maximum(inv_ref[...] * srec + b2_ref[...], 0.0)
    logits = jnp.dot(h, wfc_ref[...],
                     preferred_element_type=jnp.float32) + bfc_ref[...]
    m = jnp.max(logits, axis=1, keepdims=True)
    lse = jnp.log(jnp.sum(jnp.exp(logits - m), axis=1, keepdims=True)) + m
    out_ref[...] = logits - lse


_fin = pl.pallas_call(
    _fin_body,
    out_shape=jax.ShapeDtypeStruct((_NP, _O), jnp.float32),
)


def kernel(x, edge_index, W1, b1, W2, b2, Wfc, bfc):
    src = edge_index[0].reshape(_NW, _NCH, _CH)
    dst = edge_index[1].reshape(_NW, _NCH, _CH)
    xp = jnp.zeros((_NP, _D), x.dtype).at[:_N].set(x)

    parts = _deg_kernel(dst)
    y1, inv = _prep(parts, xp, W1)
    s1 = _agg_kernel(y1, src, dst)
    y2 = _mid(s1, y1, inv, b1.reshape(1, _H), W2)
    s2 = _agg_kernel(y2, src, dst)
    out = _fin(s2, y2, inv, b2.reshape(1, _H), Wfc, bfc.reshape(1, _O))
    return out[:_N]


# speedup vs baseline: 28.5042x; 28.5042x over previous
import functools

import jax
import jax.numpy as jnp
from jax import lax
from jax.experimental import pallas as pl
from jax.experimental.pallas import tpu as pltpu
from jax.experimental.pallas import tpu_sc as plsc

_N = 10000
_E = 320000
_D = 128
_H = 16
_O = 2

_NSUB = 16
_NCORE = 2
_NW = _NSUB * _NCORE
_NP = 10240
_RPT = _NP // _NSUB
_EPW = _E // _NW
_CH = 80
_NCH = _EPW // _CH

_mesh = plsc.VectorSubcoreMesh(core_axis_name="c", subcore_axis_name="s")
_sc_params = pltpu.CompilerParams(use_tc_tiling_on_sc=False)


@functools.partial(
    pl.kernel,
    out_type=jax.ShapeDtypeStruct((_NCORE, _NP, _H), jnp.float32),
    mesh=_mesh,
    scratch_types=[
        pltpu.VMEM((_NCH, _CH), jnp.int32),
        pltpu.VMEM((_CH, _H), jnp.float32),
        pltpu.VMEM((_RPT, _H), jnp.float32),
        pltpu.VMEM_SHARED((_NP, _H), jnp.float32),
    ],
    compiler_params=_sc_params,
)
def _deg_kernel(dstr_hbm, out_hbm, idx_v, ones_v, zbuf_v, acc_sh):
    c = lax.axis_index("c")
    s = lax.axis_index("s")
    wid = c * _NSUB + s
    zero16 = jnp.zeros((16,), jnp.float32)
    ones16 = jnp.ones((16,), jnp.float32)

    def zbody(i, carry):
        zbuf_v[i, :] = zero16
        return carry

    lax.fori_loop(0, _RPT, zbody, 0)

    def obody(i, carry):
        ones_v[i, :] = ones16
        return carry

    lax.fori_loop(0, _CH, obody, 0)
    pltpu.sync_copy(zbuf_v, acc_sh.at[pl.ds(s * _RPT, _RPT)])
    pltpu.sync_copy(dstr_hbm.at[wid], idx_v)
    plsc.subcore_barrier()

    def body(j, carry):
        pltpu.sync_copy(ones_v, acc_sh.at[idx_v.at[j]], add=True)
        return carry

    lax.fori_loop(0, _NCH, body, 0)
    plsc.subcore_barrier()
    pltpu.sync_copy(acc_sh.at[pl.ds(s * _RPT, _RPT)],
                    out_hbm.at[c, pl.ds(s * _RPT, _RPT)])


@functools.partial(
    pl.kernel,
    out_type=jax.ShapeDtypeStruct((_NCORE, _NP, _H), jnp.float32),
    mesh=_mesh,
    scratch_types=[
        pltpu.VMEM((_NCH, _CH), jnp.int32),
        pltpu.VMEM((_NCH, _CH), jnp.int32),
        pltpu.VMEM((_CH, _H), jnp.float32),
        pltpu.VMEM((_RPT, _H), jnp.float32),
        pltpu.VMEM_SHARED((_NP, _H), jnp.float32),
        pltpu.SemaphoreType.DMA,
    ],
    compiler_params=_sc_params,
)
def _agg_kernel(y_hbm, srcr_hbm, dstr_hbm, out_hbm,
                idxs_v, idxd_v, rows_v, zbuf_v, acc_sh, sem):
    c = lax.axis_index("c")
    s = lax.axis_index("s")
    wid = c * _NSUB + s
    zero16 = jnp.zeros((16,), jnp.float32)

    def zbody(i, carry):
        zbuf_v[i, :] = zero16
        return carry

    lax.fori_loop(0, _RPT, zbody, 0)
    pltpu.sync_copy(zbuf_v, acc_sh.at[pl.ds(s * _RPT, _RPT)])
    pltpu.sync_copy(srcr_hbm.at[wid], idxs_v)
    pltpu.sync_copy(dstr_hbm.at[wid], idxd_v)
    plsc.subcore_barrier()

    def body(j, carry):
        pltpu.async_copy(y_hbm.at[idxs_v.at[j]], rows_v, sem).wait()
        pltpu.sync_copy(rows_v, acc_sh.at[idxd_v.at[j]], add=True)
        return carry

    lax.fori_loop(0, _NCH, body, 0)
    plsc.subcore_barrier()
    pltpu.sync_copy(acc_sh.at[pl.ds(s * _RPT, _RPT)],
                    out_hbm.at[c, pl.ds(s * _RPT, _RPT)])


def _prep_body(parts_ref, x_ref, w1_ref, y_ref, inv_ref):
    deg = parts_ref[0, :, 0] + parts_ref[1, :, 0] + 1.0
    inv = lax.rsqrt(deg)
    xw = jnp.dot(x_ref[...], w1_ref[...], preferred_element_type=jnp.float32)
    y_ref[...] = xw * inv[:, None]
    inv_ref[...] = jnp.broadcast_to(inv[:, None], (_NP, _H))


_prep = pl.pallas_call(
    _prep_body,
    out_shape=[
        jax.ShapeDtypeStruct((_NP, _H), jnp.float32),
        jax.ShapeDtypeStruct((_NP, _H), jnp.float32),
    ],
)


def _mid_body(p_ref, y_ref, inv_ref, b_ref, w2_ref, y2_ref):
    srec = p_ref[0] + p_ref[1] + y_ref[...]
    h = jnp.maximum(inv_ref[...] * srec + b_ref[...], 0.0)
    y2_ref[...] = jnp.dot(h, w2_ref[...],
                          preferred_element_type=jnp.float32) * inv_ref[...]


_mid = pl.pallas_call(
    _mid_body,
    out_shape=jax.ShapeDtypeStruct((_NP, _H), jnp.float32),
)


def _fin_body(p_ref, y2_ref, inv_ref, b2_ref, wfc_ref, bfc_ref, out_ref):
    srec = p_ref[0] + p_ref[1] + y2_ref[...]
    h = jnp.maximum(inv_ref[...] * srec + b2_ref[...], 0.0)
    logits = jnp.dot(h, wfc_ref[...],
                     preferred_element_type=jnp.float32) + bfc_ref[...]
    m = jnp.max(logits, axis=1, keepdims=True)
    lse = jnp.log(jnp.sum(jnp.exp(logits - m), axis=1, keepdims=True)) + m
    out_ref[...] = logits - lse


_fin = pl.pallas_call(
    _fin_body,
    out_shape=jax.ShapeDtypeStruct((_NP, _O), jnp.float32),
)


def kernel(x, edge_index, W1, b1, W2, b2, Wfc, bfc):
    src = edge_index[0].reshape(_NW, _NCH, _CH)
    dst = edge_index[1].reshape(_NW, _NCH, _CH)
    xp = jnp.zeros((_NP, _D), x.dtype).at[:_N].set(x)

    parts = _deg_kernel(dst)
    y1, inv = _prep(parts, xp, W1)
    s1 = _agg_kernel(y1, src, dst)
    y2 = _mid(s1, y1, inv, b1.reshape(1, _H), W2)
    s2 = _agg_kernel(y2, src, dst)
    out = _fin(s2, y2, inv, b2.reshape(1, _H), Wfc, bfc.reshape(1, _O))
    return out[:_N]
